# baseline (speedup 1.0000x reference)
import functools

import jax
import jax.numpy as jnp
from jax import lax
from jax.experimental import pallas as pl
from jax.experimental.pallas import tpu as pltpu
from jax.experimental.pallas import tpu_sc as plsc

DIM = 64
SEN = 200
NC, NS = 2, 16
NW = NC * NS
BLK = 160
GS = 80
G = BLK // GS
PK = BLK // 16 * 8
NBUF = 2
NIB = 4


def _sc_embed(idxp, h64, Wp, pos, rows_total):
    blks = rows_total // (NW * BLK)

    @functools.partial(
        pl.kernel,
        out_type=jax.ShapeDtypeStruct((rows_total // 16 * 8, 128), jnp.float32),
        mesh=plsc.VectorSubcoreMesh(core_axis_name="c", subcore_axis_name="s"),
        scratch_types=[
            pltpu.VMEM((NIB, G, GS), jnp.int32),
            pltpu.VMEM((NIB, G, GS), jnp.int32),
            pltpu.VMEM((SEN, DIM), jnp.float32),
            pltpu.VMEM((NBUF, BLK, 128), jnp.float32),
            pltpu.VMEM((NBUF, PK, 128), jnp.float32),
        ]
        + [pltpu.SemaphoreType.DMA] * (NIB + 2 * NBUF),
    )
    def k(idx_hbm, h_hbm, w_hbm, pos_hbm, out_hbm, idx_v, h_v, pos_v, rows_v,
          stg_v, *sems):
        isem = sems[:NIB]
        gsem = sems[NIB:NIB + NBUF]
        wsem = sems[NIB + NBUF:]
        wid = lax.axis_index("s") * NC + lax.axis_index("c")
        base_pk = wid * (blks * PK)
        pltpu.async_copy(pos_hbm, pos_v, gsem[0]).wait()

        def fetch_idx(blk, ib):
            pltpu.async_copy(idx_hbm.at[wid, blk], idx_v.at[ib], isem[ib])
            pltpu.async_copy(h_hbm.at[wid, blk], h_v.at[ib], isem[ib])

        def wait_idx(ib):
            pltpu.make_async_copy(
                idx_hbm.at[wid, 0], idx_v.at[ib], isem[ib]
            ).wait()
            pltpu.make_async_copy(
                h_hbm.at[wid, 0], h_v.at[ib], isem[ib]
            ).wait()

        def start_gather(blk, ib, s):
            for g in range(G):
                pltpu.async_copy(
                    w_hbm.at[idx_v.at[ib, g]],
                    rows_v.at[s, pl.ds(g * GS, GS)],
                    gsem[s],
                )

        def wait_gather(s):
            pltpu.make_async_copy(
                w_hbm.at[pl.ds(0, BLK)], rows_v.at[s], gsem[s]
            ).wait()

        def wait_write(s):
            pltpu.make_async_copy(
                stg_v.at[s], out_hbm.at[pl.ds(0, PK)], wsem[s]
            ).wait()

        def _process(b, s, ib, u):
            wait_gather(s)

            @pl.when(b >= NBUF)
            def _():
                wait_write(s)

            phase = lax.rem(b * BLK, SEN)

            for g in range(G):
                @pl.loop(0, GS // 16)
                def _(q):
                    h16 = h_v[ib, g, pl.ds(q * 16, 16)]
                    for r16 in range(16):
                        i = g * GS + q * 16 + r16
                        a, hb = r16 % 8, r16 // 8
                        p = phase + i
                        p = jnp.where(p >= SEN, p - SEN, p)
                        h = h16[r16]
                        for c in range(DIM // 16):
                            stg_v[s, (g * (GS // 16) + q) * 8 + a,
                                  pl.ds(hb * DIM + c * 16, 16)] = (
                                rows_v[s, i, pl.ds(h + c * 16, 16)]
                                + pos_v[p, pl.ds(c * 16, 16)]
                            )

            pltpu.async_copy(
                stg_v.at[s],
                out_hbm.at[pl.ds(base_pk + b * PK, PK)],
                wsem[s],
            )

            nxt = b + 1

            @pl.when(nxt < blks)
            def _():
                wait_idx((u + 1) % NIB)
                start_gather(nxt, (u + 1) % NIB, (u + 1) % NBUF)

            pf = b + 3

            @pl.when(pf < blks)
            def _():
                fetch_idx(pf, (u + 3) % NIB)

        for b in range(min(3, blks)):
            fetch_idx(b, b % NIB)
        wait_idx(0)
        start_gather(0, 0, 0)

        @pl.loop(0, blks, step=NIB)
        def _(b0):
            for u in range(NIB):
                _process(b0 + u, u % NBUF, u, u)

        for s in range(NBUF):
            wait_write(s)

    return k(idxp, h64, Wp, pos)


def kernel(inputs, W, pos_table):
    B, S = inputs.shape
    V = W.shape[0]
    rows_total = B * S
    blks = rows_total // (NW * BLK)
    idxp = (inputs >> 1).reshape(NW, blks, G, GS)
    h64 = ((inputs & 1) << 6).reshape(NW, blks, G, GS)
    Wp = W.reshape(V // 2, 128)
    pos = pos_table[1 : S + 1]
    out = _sc_embed(idxp, h64, Wp, pos, rows_total)
    out = out.reshape(rows_total // 16, 8, 2, DIM)
    out = jnp.transpose(out, (0, 2, 1, 3))
    return out.reshape(B, S, DIM)


# speedup vs baseline: 2.2525x; 2.2525x over previous
import functools

import jax
import jax.numpy as jnp
from jax import lax
from jax.experimental import pallas as pl
from jax.experimental.pallas import tpu as pltpu
from jax.experimental.pallas import tpu_sc as plsc

DIM = 64
SEN = 200
NC, NS = 2, 16
NW = NC * NS
GSZ = 100
GPB = SEN // GSZ


def _sc_embed(idx, W, pos, blks_per_w):
    NBUF = 4
    LOOK = 2

    @functools.partial(
        pl.kernel,
        out_type=jax.ShapeDtypeStruct((NW, blks_per_w, SEN, DIM), jnp.float32),
        mesh=plsc.VectorSubcoreMesh(core_axis_name="c", subcore_axis_name="s"),
        scratch_types=[
            pltpu.VMEM((blks_per_w * GPB, GSZ), jnp.int32),
            pltpu.VMEM((SEN, DIM), jnp.float32),
            pltpu.VMEM((NBUF, SEN, DIM), jnp.float32),
        ]
        + [pltpu.SemaphoreType.DMA] * (2 * NBUF),
        compiler_params=pltpu.CompilerParams(use_tc_tiling_on_sc=False),
    )
    def k(idx_hbm, w_hbm, pos_hbm, out_hbm, idx_v, pos_v, rows_v, *sems):
        gsem, wsem = sems[:NBUF], sems[NBUF:]
        wid = lax.axis_index("s") * NC + lax.axis_index("c")
        pltpu.async_copy(idx_hbm.at[wid], idx_v, gsem[0]).wait()
        pltpu.async_copy(pos_hbm, pos_v, gsem[0]).wait()

        def start_gather(blk, slot):
            for h in range(GPB):
                pltpu.async_copy(
                    w_hbm.at[idx_v.at[blk * GPB + h]],
                    rows_v.at[slot, pl.ds(h * GSZ, GSZ)],
                    gsem[slot],
                )

        def wait_gather(slot):
            pltpu.make_async_copy(
                w_hbm.at[pl.ds(0, SEN)], rows_v.at[slot], gsem[slot]
            ).wait()

        def wait_write(slot):
            pltpu.make_async_copy(
                rows_v.at[slot], out_hbm.at[wid, 0], wsem[slot]
            ).wait()

        for b in range(LOOK):
            start_gather(b, b)

        @pl.loop(0, blks_per_w, step=NBUF)
        def _(b0):
            for s in range(NBUF):
                blk = b0 + s
                wait_gather(s)

                @pl.loop(0, SEN)
                def _(i):
                    for j in range(DIM // 16):
                        sl = pl.ds(j * 16, 16)
                        rows_v[s, i, sl] = rows_v[s, i, sl] + pos_v[i, sl]

                pltpu.async_copy(rows_v.at[s], out_hbm.at[wid, blk], wsem[s])

                gblk = blk + LOOK
                gslot = (s + LOOK) % NBUF

                @pl.when(gblk < blks_per_w)
                def _():
                    @pl.when(gblk >= NBUF)
                    def _():
                        wait_write(gslot)

                    start_gather(gblk, gslot)

        for s in range(NBUF):
            wait_write(s)

    return k(idx, W, pos)


def kernel(inputs, W, pos_table):
    B, S = inputs.shape
    blks_per_w = (B * S) // (NW * SEN)
    idx = inputs.reshape(NW, blks_per_w * GPB, GSZ)
    pos = pos_table[1 : S + 1]
    out = _sc_embed(idx, W, pos, blks_per_w)
    return out.reshape(B, S, DIM)
